# baseline (speedup 1.0000x reference)
import jax
import jax.numpy as jnp
from jax.experimental import pallas as pl


def _broadcast_body(emb_ref, out_ref):
    out_ref[0] = emb_ref[...]


def kernel(x, grid_embedding):
    batch = x.shape[0]
    g2, f = grid_embedding.shape
    return pl.pallas_call(
        _broadcast_body,
        grid=(batch,),
        in_specs=[pl.BlockSpec((g2, f), lambda b: (0, 0))],
        out_specs=pl.BlockSpec((1, g2, f), lambda b: (b, 0, 0)),
        out_shape=jax.ShapeDtypeStruct((batch, g2, f), grid_embedding.dtype),
    )(grid_embedding)


# speedup vs baseline: 1.0002x; 1.0002x over previous
import jax
import jax.numpy as jnp
from jax.experimental import pallas as pl


from jax.experimental.pallas import tpu as pltpu


def _make_broadcast_body(batch):
    def body(emb_any, out_any, scratch, load_sem, sems):
        cp = pltpu.make_async_copy(emb_any, scratch, load_sem)
        cp.start()
        cp.wait()
        for b in range(batch):
            pltpu.make_async_copy(scratch, out_any.at[b], sems.at[b]).start()
        for b in range(batch):
            pltpu.make_async_copy(scratch, out_any.at[b], sems.at[b]).wait()
    return body


def kernel(x, grid_embedding):
    batch = x.shape[0]
    g2, f = grid_embedding.shape
    return pl.pallas_call(
        _make_broadcast_body(batch),
        in_specs=[pl.BlockSpec(memory_space=pl.ANY)],
        out_specs=pl.BlockSpec(memory_space=pl.ANY),
        out_shape=jax.ShapeDtypeStruct((batch, g2, f), grid_embedding.dtype),
        scratch_shapes=[
            pltpu.VMEM((g2, f), grid_embedding.dtype),
            pltpu.SemaphoreType.DMA,
            pltpu.SemaphoreType.DMA((batch,)),
        ],
    )(grid_embedding)
